# baseline (speedup 1.0000x reference)
import functools

import jax
import jax.numpy as jnp
from jax import lax
from jax.experimental import pallas as pl
from jax.experimental.pallas import tpu as pltpu
from jax.experimental.pallas import tpu_sc as plsc

H, W = 50, 50
LANES = 16
QB = 240


def _c(v):
  return jnp.full((LANES,), v, jnp.int32)


def _deform_body(nq, nh, ed, npts, bh_per_worker, nblocks, num_cores,
                 val_ref, loc_ref, att_ref, out_ref,
                 table_v, loc_v, att_v, out_v, lsem, asem, osem):
  wid = lax.axis_index("s") * num_cores + lax.axis_index("c")
  lanes = lax.iota(jnp.int32, LANES)

  @pl.loop(0, bh_per_worker)
  def _bh_loop(t):
    bh = wid * bh_per_worker + t
    b = bh // nh
    h = bh % nh

    def in_copies(blk, slot):
      q0 = jnp.minimum(blk * QB, nq - QB)
      return (
          pltpu.make_async_copy(
              loc_ref.at[b, pl.ds(q0, QB), h, 0], loc_v.at[slot],
              lsem.at[slot]),
          pltpu.make_async_copy(
              att_ref.at[b, pl.ds(q0, QB), h, 0], att_v.at[slot],
              asem.at[slot]),
      )

    def out_copy(blk, slot):
      q0 = jnp.minimum(blk * QB, nq - QB)
      return pltpu.make_async_copy(
          out_v.at[slot], out_ref.at[b, pl.ds(q0, QB), pl.ds(h * ed, ed)],
          osem.at[slot])

    for c in in_copies(0, 0):
      c.start()
    pltpu.sync_copy(val_ref.at[b, :, h, :], table_v)

    @pl.loop(0, nblocks)
    def _block_loop(blk):
      slot = blk % 2

      @pl.when(blk + 1 < nblocks)
      def _start_next():
        for c in in_copies(blk + 1, 1 - slot):
          c.start()

      for c in in_copies(blk, slot):
        c.wait()
      @pl.when(blk >= 2)
      def _drain_out():
        out_copy(blk - 2, slot).wait()

      slot_v = jnp.full((LANES,), slot, jnp.int32)

      @plsc.parallel_loop(0, QB // LANES)
      def _chunk_loop(ck):
        qv = ck * LANES + lanes
        for p in range(npts):
          lx = plsc.load_gather(loc_v, [slot_v, qv, _c(p), _c(0)])
          ly = plsc.load_gather(loc_v, [slot_v, qv, _c(p), _c(1)])
          aw = plsc.load_gather(att_v, [slot_v, qv, _c(p)])
          x = lx * float(W) - 0.5
          y = ly * float(H) - 0.5
          x0 = (x + 1.0).astype(jnp.int32) - 1
          y0 = (y + 1.0).astype(jnp.int32) - 1
          fx = x - x0.astype(jnp.float32)
          fy = y - y0.astype(jnp.float32)
          x1 = x0 + 1
          y1 = y0 + 1
          wx = (1.0 - fx, fx)
          wy = (1.0 - fy, fy)
          vx = (x0 >= 0, x1 <= W - 1)
          vy = (y0 >= 0, y1 <= H - 1)
          cx = (jnp.clip(x0, 0, W - 1), jnp.clip(x1, 0, W - 1))
          cy = (jnp.clip(y0, 0, H - 1), jnp.clip(y1, 0, H - 1))
          for iy in range(2):
            for ix in range(2):
              wgt = jnp.where(vx[ix] & vy[iy], wx[ix] * wy[iy] * aw, 0.0)
              row = cy[iy] * W + cx[ix]
              for f in range(ed):
                v = wgt * plsc.load_gather(table_v, [row, _c(f)])
                if p == 0 and iy == 0 and ix == 0:
                  plsc.store_scatter(out_v, [slot_v, qv, _c(f)], v)
                else:
                  plsc.addupdate_scatter(out_v, [slot_v, qv, _c(f)], v)

      out_copy(blk, slot).start()

    out_copy(nblocks - 2, (nblocks - 2) % 2).wait()
    out_copy(nblocks - 1, (nblocks - 1) % 2).wait()


def kernel(value, value_spatial_shapes, sampling_locations, attention_weights):
  del value_spatial_shapes
  bs, nk, nh, ed = value.shape
  nq = sampling_locations.shape[1]
  npts = sampling_locations.shape[4]

  info = plsc.get_sparse_core_info()
  num_cores, num_subcores = info.num_cores, info.num_subcores
  nworkers = num_cores * num_subcores
  assert (bs * nh) % nworkers == 0
  bh_per_worker = (bs * nh) // nworkers
  nblocks = (nq + QB - 1) // QB

  mesh = plsc.VectorSubcoreMesh(core_axis_name="c", subcore_axis_name="s")
  body = functools.partial(_deform_body, nq, nh, ed, npts, bh_per_worker,
                           nblocks, num_cores)
  out = pl.kernel(
      body,
      out_type=jax.ShapeDtypeStruct((bs, nq, nh * ed), jnp.float32),
      mesh=mesh,
      compiler_params=pltpu.CompilerParams(
          needs_layout_passes=False, use_tc_tiling_on_sc=False),
      scratch_types=[
          pltpu.VMEM((nk, ed), jnp.float32),
          pltpu.VMEM((2, QB, npts, 2), jnp.float32),
          pltpu.VMEM((2, QB, npts), jnp.float32),
          pltpu.VMEM((2, QB, ed), jnp.float32),
          pltpu.SemaphoreType.DMA((2,)),
          pltpu.SemaphoreType.DMA((2,)),
          pltpu.SemaphoreType.DMA((2,)),
      ],
  )(value, sampling_locations, attention_weights)
  return out


# speedup vs baseline: 16.6147x; 16.6147x over previous
import functools

import jax
import jax.numpy as jnp
from jax import lax
from jax.experimental import pallas as pl
from jax.experimental.pallas import tpu as pltpu
from jax.experimental.pallas import tpu_sc as plsc

H, W = 50, 50
LANES = 16
QB = 240


def _c(v):
  return jnp.full((LANES,), v, jnp.int32)


def _deform_body(nq, nh, ed, npts, bh_per_worker, nblocks, num_cores,
                 val_ref, loc_ref, att_ref, out_ref,
                 table_v, loc_v, att_v, out_v, lsem, asem, osem):
  wid = lax.axis_index("s") * num_cores + lax.axis_index("c")
  lanes = lax.iota(jnp.int32, LANES)

  @pl.loop(0, bh_per_worker)
  def _bh_loop(t):
    bh = wid * bh_per_worker + t
    b = bh // nh
    h = bh % nh

    def in_copies(blk, slot):
      q0 = jnp.minimum(blk * QB, nq - QB)
      return (
          pltpu.make_async_copy(
              loc_ref.at[b, h, :, :, pl.ds(q0, QB)], loc_v.at[slot],
              lsem.at[slot]),
          pltpu.make_async_copy(
              att_ref.at[b, h, :, pl.ds(q0, QB)], att_v.at[slot],
              asem.at[slot]),
      )

    def out_copy(blk, slot):
      q0 = jnp.minimum(blk * QB, nq - QB)
      return pltpu.make_async_copy(
          out_v.at[slot], out_ref.at[b, h, :, pl.ds(q0, QB)],
          osem.at[slot])

    for c in in_copies(0, 0):
      c.start()
    pltpu.sync_copy(val_ref.at[b, h], table_v)

    @pl.loop(0, nblocks)
    def _block_loop(blk):
      slot = blk % 2

      @pl.when(blk + 1 < nblocks)
      def _start_next():
        for c in in_copies(blk + 1, 1 - slot):
          c.start()

      for c in in_copies(blk, slot):
        c.wait()
      @pl.when(blk >= 2)
      def _drain_out():
        out_copy(blk - 2, slot).wait()

      @plsc.parallel_loop(0, QB // LANES)
      def _chunk_loop(ck):
        qs = pl.ds(ck * LANES, LANES)
        lxs = [loc_v[slot, p, 0, qs] for p in range(npts)]
        lys = [loc_v[slot, p, 1, qs] for p in range(npts)]
        aws = [att_v[slot, p, qs] for p in range(npts)]
        for p in range(npts):
          lx, ly, aw = lxs[p], lys[p], aws[p]
          x = lx * float(W) - 0.5
          y = ly * float(H) - 0.5
          x0 = (x + 1.0).astype(jnp.int32) - 1
          y0 = (y + 1.0).astype(jnp.int32) - 1
          fx = x - x0.astype(jnp.float32)
          fy = y - y0.astype(jnp.float32)
          x1 = x0 + 1
          y1 = y0 + 1
          wx = (1.0 - fx, fx)
          wy = (1.0 - fy, fy)
          vx = (x0 >= 0, x1 <= W - 1)
          vy = (y0 >= 0, y1 <= H - 1)
          cx = (jnp.clip(x0, 0, W - 1), jnp.clip(x1, 0, W - 1))
          cy = (jnp.clip(y0, 0, H - 1), jnp.clip(y1, 0, H - 1))
          for iy in range(2):
            for ix in range(2):
              wgt = jnp.where(vx[ix] & vy[iy], wx[ix] * wy[iy] * aw, 0.0)
              row = cy[iy] * W + cx[ix]
              first = p == 0 and iy == 0 and ix == 0
              for g in range(0, ed, 8):
                fs = range(g, g + 8)
                vals = [plsc.load_gather(table_v, [_c(f), row]) for f in fs]
                prods = [wgt * v for v in vals]
                for j, f in enumerate(fs):
                  if first:
                    out_v[slot, f, qs] = prods[j]
                  else:
                    plsc.addupdate(out_v.at[slot, f, qs], prods[j])

      out_copy(blk, slot).start()

    out_copy(nblocks - 2, (nblocks - 2) % 2).wait()
    out_copy(nblocks - 1, (nblocks - 1) % 2).wait()


def kernel(value, value_spatial_shapes, sampling_locations, attention_weights):
  del value_spatial_shapes
  bs, nk, nh, ed = value.shape
  nq = sampling_locations.shape[1]
  npts = sampling_locations.shape[4]

  loc_t = jnp.transpose(
      sampling_locations.reshape(bs, nq, nh, npts, 2), (0, 2, 3, 4, 1))
  att_t = jnp.transpose(
      attention_weights.reshape(bs, nq, nh, npts), (0, 2, 3, 1))
  val_t = jnp.transpose(value, (0, 2, 3, 1))

  info = plsc.get_sparse_core_info()
  num_cores, num_subcores = info.num_cores, info.num_subcores
  nworkers = num_cores * num_subcores
  assert (bs * nh) % nworkers == 0
  bh_per_worker = (bs * nh) // nworkers
  nblocks = (nq + QB - 1) // QB

  mesh = plsc.VectorSubcoreMesh(core_axis_name="c", subcore_axis_name="s")
  body = functools.partial(_deform_body, nq, nh, ed, npts, bh_per_worker,
                           nblocks, num_cores)
  out = pl.kernel(
      body,
      out_type=jax.ShapeDtypeStruct((bs, nh, ed, nq), jnp.float32),
      mesh=mesh,
      compiler_params=pltpu.CompilerParams(
          needs_layout_passes=False, use_tc_tiling_on_sc=False),
      scratch_types=[
          pltpu.VMEM((ed, nk), jnp.float32),
          pltpu.VMEM((2, npts, 2, QB), jnp.float32),
          pltpu.VMEM((2, npts, QB), jnp.float32),
          pltpu.VMEM((2, ed, QB), jnp.float32),
          pltpu.SemaphoreType.DMA((2,)),
          pltpu.SemaphoreType.DMA((2,)),
          pltpu.SemaphoreType.DMA((2,)),
      ],
  )(val_t, loc_t, att_t)
  return jnp.transpose(out, (0, 3, 1, 2)).reshape(bs, nq, nh * ed)


# speedup vs baseline: 21.4074x; 1.2885x over previous
import functools

import jax
import jax.numpy as jnp
from jax import lax
from jax.experimental import pallas as pl
from jax.experimental.pallas import tpu as pltpu
from jax.experimental.pallas import tpu_sc as plsc

H, W = 50, 50
LANES = 16
QB = 240


def _c(v):
  return jnp.full((LANES,), v, jnp.int32)


def _deform_body(nq, nh, ed, npts, bh_per_worker, nblocks, num_cores,
                 val_ref, loc_ref, att_ref, out_ref,
                 table_v, loc_v, att_v, out_v, lsem, asem, osem):
  wid = lax.axis_index("s") * num_cores + lax.axis_index("c")
  lanes = lax.iota(jnp.int32, LANES)

  @pl.loop(0, bh_per_worker)
  def _bh_loop(t):
    bh = wid * bh_per_worker + t
    b = bh // nh
    h = bh % nh

    def in_copies(blk, slot):
      q0 = jnp.minimum(blk * QB, nq - QB)
      return (
          pltpu.make_async_copy(
              loc_ref.at[b, h, :, :, pl.ds(q0, QB)], loc_v.at[slot],
              lsem.at[slot]),
          pltpu.make_async_copy(
              att_ref.at[b, h, :, pl.ds(q0, QB)], att_v.at[slot],
              asem.at[slot]),
      )

    def out_copy(blk, slot):
      q0 = jnp.minimum(blk * QB, nq - QB)
      return pltpu.make_async_copy(
          out_v.at[slot], out_ref.at[b, h, :, pl.ds(q0, QB)],
          osem.at[slot])

    for c in in_copies(0, 0):
      c.start()
    pltpu.sync_copy(val_ref.at[b, h], table_v)

    @pl.loop(0, nblocks)
    def _block_loop(blk):
      slot = blk % 2

      @pl.when(blk + 1 < nblocks)
      def _start_next():
        for c in in_copies(blk + 1, 1 - slot):
          c.start()

      for c in in_copies(blk, slot):
        c.wait()
      @pl.when(blk >= 2)
      def _drain_out():
        out_copy(blk - 2, slot).wait()

      @plsc.parallel_loop(0, QB // LANES)
      def _chunk_loop(ck):
        qs = pl.ds(ck * LANES, LANES)
        for half in range(0, ed, ed // 2):
          lxs = [loc_v[slot, p, 0, qs] for p in range(npts)]
          lys = [loc_v[slot, p, 1, qs] for p in range(npts)]
          aws = [att_v[slot, p, qs] for p in range(npts)]
          accs = [None] * (ed // 2)
          for p in range(npts):
            lx, ly, aw = lxs[p], lys[p], aws[p]
            x = lx * float(W) - 0.5
            y = ly * float(H) - 0.5
            x0 = (x + 1.0).astype(jnp.int32) - 1
            y0 = (y + 1.0).astype(jnp.int32) - 1
            fx = x - x0.astype(jnp.float32)
            fy = y - y0.astype(jnp.float32)
            x1 = x0 + 1
            y1 = y0 + 1
            wx = (1.0 - fx, fx)
            wy = (1.0 - fy, fy)
            vx = (x0 >= 0, x1 <= W - 1)
            vy = (y0 >= 0, y1 <= H - 1)
            cx = (jnp.clip(x0, 0, W - 1), jnp.clip(x1, 0, W - 1))
            cy = (jnp.clip(y0, 0, H - 1), jnp.clip(y1, 0, H - 1))
            for iy in range(2):
              for ix in range(2):
                wgt = jnp.where(vx[ix] & vy[iy], wx[ix] * wy[iy] * aw, 0.0)
                row = cy[iy] * W + cx[ix]
                for g in range(half, half + ed // 2, 8):
                  fs = range(g, g + 8)
                  vals = [plsc.load_gather(table_v, [_c(f), row])
                          for f in fs]
                  for j, f in enumerate(fs):
                    k = f - half
                    v = wgt * vals[j]
                    accs[k] = v if accs[k] is None else accs[k] + v
          for k in range(ed // 2):
            out_v[slot, half + k, qs] = accs[k]

      out_copy(blk, slot).start()

    out_copy(nblocks - 2, (nblocks - 2) % 2).wait()
    out_copy(nblocks - 1, (nblocks - 1) % 2).wait()


def kernel(value, value_spatial_shapes, sampling_locations, attention_weights):
  del value_spatial_shapes
  bs, nk, nh, ed = value.shape
  nq = sampling_locations.shape[1]
  npts = sampling_locations.shape[4]

  loc_t = jnp.transpose(
      sampling_locations.reshape(bs, nq, nh, npts, 2), (0, 2, 3, 4, 1))
  att_t = jnp.transpose(
      attention_weights.reshape(bs, nq, nh, npts), (0, 2, 3, 1))
  val_t = jnp.transpose(value, (0, 2, 3, 1))

  info = plsc.get_sparse_core_info()
  num_cores, num_subcores = info.num_cores, info.num_subcores
  nworkers = num_cores * num_subcores
  assert (bs * nh) % nworkers == 0
  bh_per_worker = (bs * nh) // nworkers
  nblocks = (nq + QB - 1) // QB

  mesh = plsc.VectorSubcoreMesh(core_axis_name="c", subcore_axis_name="s")
  body = functools.partial(_deform_body, nq, nh, ed, npts, bh_per_worker,
                           nblocks, num_cores)
  out = pl.kernel(
      body,
      out_type=jax.ShapeDtypeStruct((bs, nh, ed, nq), jnp.float32),
      mesh=mesh,
      compiler_params=pltpu.CompilerParams(
          needs_layout_passes=False, use_tc_tiling_on_sc=False),
      scratch_types=[
          pltpu.VMEM((ed, nk), jnp.float32),
          pltpu.VMEM((2, npts, 2, QB), jnp.float32),
          pltpu.VMEM((2, npts, QB), jnp.float32),
          pltpu.VMEM((2, ed, QB), jnp.float32),
          pltpu.SemaphoreType.DMA((2,)),
          pltpu.SemaphoreType.DMA((2,)),
          pltpu.SemaphoreType.DMA((2,)),
      ],
  )(val_t, loc_t, att_t)
  return jnp.transpose(out, (0, 3, 1, 2)).reshape(bs, nq, nh * ed)
